# baseline (speedup 1.0000x reference)
import functools

import jax
import jax.numpy as jnp
from jax import lax
from jax.experimental import pallas as pl
from jax.experimental.pallas import tpu as pltpu
from jax.experimental.pallas import tpu_sc as plsc

N = 10000
E = 320000
H = 128
NG = 64
F1 = 54
F2 = 18
MID = 64

NB = 2000
EB = 2000
NPAD = 10240
ROWS_PER_TILE = NPAD // 16
EDGES_PER_TILE = E // 16
CHUNK = 40
NCHUNK = EDGES_PER_TILE // CHUNK

_P = jax.lax.Precision.HIGHEST


def _swish(v):
    return v * jax.nn.sigmoid(v)


def _dot(a, b):
    return jnp.dot(a, b, precision=_P, preferred_element_type=jnp.float32)


def _xprime_body(x_ref, w_ref, b_ref, o_ref):
    o_ref[...] = _swish(_dot(x_ref[...], w_ref[...]) + b_ref[...])


def _xprime(x, W, b):
    return pl.pallas_call(
        _xprime_body,
        grid=(N // NB,),
        in_specs=[
            pl.BlockSpec((NB, H), lambda i: (i, 0)),
            pl.BlockSpec((H, H), lambda i: (0, 0)),
            pl.BlockSpec((1, H), lambda i: (0, 0)),
        ],
        out_specs=pl.BlockSpec((NB, H), lambda i: (i, 0)),
        out_shape=jax.ShapeDtypeStruct((N, H), jnp.float32),
    )(x, W, b)


def _edgefeat_body(f_ref, wa_ref, wb_ref, o_ref):
    o_ref[...] = _dot(_dot(f_ref[...], wa_ref[...]), wb_ref[...])


def _edgefeat(feature, Wfa, Wfb):
    fdim = feature.shape[1]
    return pl.pallas_call(
        _edgefeat_body,
        grid=(E // EB,),
        in_specs=[
            pl.BlockSpec((EB, fdim), lambda i: (i, 0)),
            pl.BlockSpec((fdim, MID), lambda i: (0, 0)),
            pl.BlockSpec((MID, H), lambda i: (0, 0)),
        ],
        out_specs=pl.BlockSpec((EB, H), lambda i: (i, 0)),
        out_shape=jax.ShapeDtypeStruct((E, H), jnp.float32),
    )(feature, Wfa, Wfb)


def _sc_conv(xp, f, src3, dst3, zeros):
    mesh = plsc.VectorSubcoreMesh(core_axis_name="c", subcore_axis_name="s",
                                  num_cores=1)

    @functools.partial(
        pl.kernel,
        mesh=mesh,
        out_type=jax.ShapeDtypeStruct((NPAD, H), jnp.float32),
        scratch_types=[
            pltpu.VMEM((CHUNK,), jnp.int32),
            pltpu.VMEM((CHUNK,), jnp.int32),
            pltpu.VMEM((CHUNK,), jnp.int32),
            pltpu.VMEM((CHUNK,), jnp.int32),
            pltpu.VMEM((CHUNK, H), jnp.float32),
            pltpu.VMEM((CHUNK, H), jnp.float32),
            pltpu.VMEM((CHUNK, H), jnp.float32),
            pltpu.VMEM((CHUNK, H), jnp.float32),
            pltpu.VMEM((CHUNK, H), jnp.float32),
            pltpu.VMEM((CHUNK, H), jnp.float32),
            pltpu.VMEM_SHARED((NPAD, H), jnp.float32),
            pltpu.SemaphoreType.DMA,
            pltpu.SemaphoreType.DMA,
            pltpu.SemaphoreType.DMA,
            pltpu.SemaphoreType.DMA,
            pltpu.SemaphoreType.DMA,
            pltpu.SemaphoreType.DMA,
            pltpu.SemaphoreType.DMA,
            pltpu.SemaphoreType.DMA,
        ],
    )
    def k(x_hbm, f_hbm, src_hbm, dst_hbm, z_hbm, agg_hbm,
          sv0, sv1, dv0, dv1, xr0, xr1, fb0, fb1, pr0, pr1, agg_sh,
          sG0, sG1, sF0, sF1, sSI0, sSI1, sDI0, sDI1):
        s = lax.axis_index("s")
        sv = (sv0, sv1)
        dv = (dv0, dv1)
        xr = (xr0, xr1)
        fb = (fb0, fb1)
        pr = (pr0, pr1)
        sG = (sG0, sG1)
        sF = (sF0, sF1)
        sSI = (sSI0, sSI1)
        sDI = (sDI0, sDI1)
        nbase = s * ROWS_PER_TILE
        ebase = s * EDGES_PER_TILE

        def start_data(j, b):
            pltpu.async_copy(x_hbm.at[sv[b]], xr[b], sG[b])
            pltpu.async_copy(f_hbm.at[pl.ds(ebase + j * CHUNK, CHUNK)],
                             fb[b], sF[b])

        for b in range(2):
            pltpu.sync_copy(src_hbm.at[s, b], sv[b])
            pltpu.sync_copy(dst_hbm.at[s, b], dv[b])
            start_data(b, b)

        pltpu.sync_copy(z_hbm.at[pl.ds(nbase, ROWS_PER_TILE)],
                        agg_sh.at[pl.ds(nbase, ROWS_PER_TILE)])
        plsc.subcore_barrier()

        def pair(g, carry):
            for b in range(2):
                j = g * 2 + b
                pltpu.make_async_copy(x_hbm.at[sv[b]], xr[b], sG[b]).wait()

                @pl.when(g < NCHUNK // 2 - 1)
                def _():
                    pltpu.async_copy(src_hbm.at[s, j + 2], sv[b], sSI[b])

                pltpu.make_async_copy(
                    f_hbm.at[pl.ds(ebase + j * CHUNK, CHUNK)], fb[b],
                    sF[b]).wait()

                @pl.when(g >= 1)
                def _():
                    pltpu.make_async_copy(dst_hbm.at[s, j], dv[b], sDI[b]).wait()

                @plsc.parallel_loop(0, CHUNK, unroll=4)
                def _(r):
                    for kk in range(H // 16):
                        sl = pl.ds(kk * 16, 16)
                        pr[b][r, sl] = fb[b][r, sl] * xr[b][r, sl]

                pltpu.sync_copy(pr[b], agg_sh.at[dv[b]], add=True)

                @pl.when(g < NCHUNK // 2 - 1)
                def _():
                    pltpu.async_copy(dst_hbm.at[s, j + 2], dv[b], sDI[b])
                    pltpu.make_async_copy(src_hbm.at[s, j + 2], sv[b],
                                          sSI[b]).wait()
                    start_data(j + 2, b)
            return carry

        lax.fori_loop(0, NCHUNK // 2, pair, 0)
        plsc.subcore_barrier()
        pltpu.sync_copy(agg_sh.at[pl.ds(nbase, ROWS_PER_TILE)],
                        agg_hbm.at[pl.ds(nbase, ROWS_PER_TILE)])

    return k(xp, f, src3, dst3, zeros)


def _b1_body(agg1_ref, agg2_ref, x_ref, bid_ref,
             wrel1_ref, brel1_ref, wroot1_ref, w1_ref, b1_ref,
             wrel2_ref, brel2_ref, wroot2_ref, w2_ref, b2_ref,
             wcat_ref, bcat_ref, wl0_ref, bl0_ref, wl1_ref, bl1_ref,
             hpre_ref, gsum_ref, gcnt_ref):
    xb = x_ref[...]
    h1 = _dot(agg1_ref[...], wrel1_ref[...]) + brel1_ref[...] + _dot(xb, wroot1_ref[...])
    h1 = _swish(_dot(h1, w1_ref[...]) + b1_ref[...])
    h2 = _dot(agg2_ref[...], wrel2_ref[...]) + brel2_ref[...] + _dot(xb, wroot2_ref[...])
    h2 = _swish(_dot(h2, w2_ref[...]) + b2_ref[...])
    h = _dot(h1, wcat_ref[...][:H]) + _dot(h2, wcat_ref[...][H:]) + bcat_ref[...] + xb
    h = _swish(_dot(h, wl0_ref[...]) + bl0_ref[...]) + h
    h = _swish(_dot(h, wl1_ref[...]) + bl1_ref[...]) + h
    hpre_ref[...] = h
    ids = bid_ref[0]
    oh = (lax.broadcasted_iota(jnp.int32, (NG, NB), 0) == ids).astype(jnp.float32)
    psum = _dot(oh, h)
    pcnt = jnp.broadcast_to(jnp.sum(oh, axis=1, keepdims=True), (NG, H))

    @pl.when(pl.program_id(0) == 0)
    def _():
        gsum_ref[...] = psum
        gcnt_ref[...] = pcnt

    @pl.when(pl.program_id(0) != 0)
    def _():
        gsum_ref[...] += psum
        gcnt_ref[...] += pcnt


def _b1(agg1, agg2, xp, bid_row, p):
    wspec = pl.BlockSpec((H, H), lambda i: (0, 0))
    bspec = pl.BlockSpec((1, H), lambda i: (0, 0))
    return pl.pallas_call(
        _b1_body,
        grid=(N // NB,),
        in_specs=[
            pl.BlockSpec((NB, H), lambda i: (i, 0)),
            pl.BlockSpec((NB, H), lambda i: (i, 0)),
            pl.BlockSpec((NB, H), lambda i: (i, 0)),
            pl.BlockSpec((1, 1, NB), lambda i: (i, 0, 0)),
            wspec, bspec, wspec, wspec, bspec,
            wspec, bspec, wspec, wspec, bspec,
            pl.BlockSpec((2 * H, H), lambda i: (0, 0)), bspec,
            wspec, bspec, wspec, bspec,
        ],
        out_specs=[
            pl.BlockSpec((NB, H), lambda i: (i, 0)),
            pl.BlockSpec((NG, H), lambda i: (0, 0)),
            pl.BlockSpec((NG, H), lambda i: (0, 0)),
        ],
        out_shape=[
            jax.ShapeDtypeStruct((N, H), jnp.float32),
            jax.ShapeDtypeStruct((NG, H), jnp.float32),
            jax.ShapeDtypeStruct((NG, H), jnp.float32),
        ],
    )(agg1, agg2, xp, bid_row,
      p["Wrel1"], p["brel1"].reshape(1, H), p["Wroot1"], p["W1"], p["b1"].reshape(1, H),
      p["Wrel2"], p["brel2"].reshape(1, H), p["Wroot2"], p["W2"], p["b2"].reshape(1, H),
      p["Wcat"], p["bcat"].reshape(1, H), p["Wl0"], p["bl0"].reshape(1, H),
      p["Wl1"], p["bl1"].reshape(1, H))


def _b2_body(h_ref, bidr_ref, bidc_ref, gsum_ref, gcnt_ref, ms_ref, vsum_ref):
    h = h_ref[...]
    cnt = jnp.maximum(gcnt_ref[...], 1.0)
    mean = gsum_ref[...] / cnt
    idc = bidc_ref[...]
    ohc = (lax.broadcasted_iota(jnp.int32, (NB, NG), 1) == idc).astype(jnp.float32)
    cen = h - _dot(ohc, mean) * ms_ref[...]
    idr = bidr_ref[0]
    ohr = (lax.broadcasted_iota(jnp.int32, (NG, NB), 0) == idr).astype(jnp.float32)
    pv = _dot(ohr, cen * cen)

    @pl.when(pl.program_id(0) == 0)
    def _():
        vsum_ref[...] = pv

    @pl.when(pl.program_id(0) != 0)
    def _():
        vsum_ref[...] += pv


def _b2(hpre, bid_row, bid_col, gsum, gcnt, norm_ms):
    return pl.pallas_call(
        _b2_body,
        grid=(N // NB,),
        in_specs=[
            pl.BlockSpec((NB, H), lambda i: (i, 0)),
            pl.BlockSpec((1, 1, NB), lambda i: (i, 0, 0)),
            pl.BlockSpec((NB, 1), lambda i: (i, 0)),
            pl.BlockSpec((NG, H), lambda i: (0, 0)),
            pl.BlockSpec((NG, H), lambda i: (0, 0)),
            pl.BlockSpec((1, H), lambda i: (0, 0)),
        ],
        out_specs=pl.BlockSpec((NG, H), lambda i: (0, 0)),
        out_shape=jax.ShapeDtypeStruct((NG, H), jnp.float32),
    )(hpre, bid_row, bid_col, gsum, gcnt, norm_ms)


def _b3_body(h_ref, bidc_ref, gsum_ref, gcnt_ref, vsum_ref,
             nw_ref, nb_ref, ms_ref, wfin_ref, bfin_ref, o_ref):
    cnt = jnp.maximum(gcnt_ref[...], 1.0)
    mean = gsum_ref[...] / cnt
    std = jnp.sqrt(vsum_ref[...] / cnt + 1e-5)
    idc = bidc_ref[...]
    ohc = (lax.broadcasted_iota(jnp.int32, (NB, NG), 1) == idc).astype(jnp.float32)
    cen = h_ref[...] - _dot(ohc, mean) * ms_ref[...]
    hn = nw_ref[...] * cen / _dot(ohc, std) + nb_ref[...]
    o_ref[...] = _dot(hn, wfin_ref[...]) + bfin_ref[...]


def _b3(hpre, bid_col, gsum, gcnt, vsum, p):
    return pl.pallas_call(
        _b3_body,
        grid=(N // NB,),
        in_specs=[
            pl.BlockSpec((NB, H), lambda i: (i, 0)),
            pl.BlockSpec((NB, 1), lambda i: (i, 0)),
            pl.BlockSpec((NG, H), lambda i: (0, 0)),
            pl.BlockSpec((NG, H), lambda i: (0, 0)),
            pl.BlockSpec((NG, H), lambda i: (0, 0)),
            pl.BlockSpec((1, H), lambda i: (0, 0)),
            pl.BlockSpec((1, H), lambda i: (0, 0)),
            pl.BlockSpec((1, H), lambda i: (0, 0)),
            pl.BlockSpec((H, H), lambda i: (0, 0)),
            pl.BlockSpec((1, H), lambda i: (0, 0)),
        ],
        out_specs=pl.BlockSpec((NB, H), lambda i: (i, 0)),
        out_shape=jax.ShapeDtypeStruct((N, H), jnp.float32),
    )(hpre, bid_col, gsum, gcnt, vsum,
      p["norm_w"].reshape(1, H), p["norm_b"].reshape(1, H),
      p["norm_ms"].reshape(1, H), p["Wfin"], p["bfin"].reshape(1, H))


def kernel(x, feature1, feature2, edge_index, batch, params):
    p = params
    ei = edge_index.astype(jnp.int32)
    src = ei[0].reshape(16, NCHUNK, CHUNK)
    dst = ei[1].reshape(16, NCHUNK, CHUNK)
    bid = batch.astype(jnp.int32)
    bid_row = bid.reshape(N // NB, 1, NB)
    bid_col = bid.reshape(N, 1)

    xp = _xprime(x, p["W_lin"], p["b_lin"].reshape(1, H))
    zeros = jnp.zeros((NPAD, H), jnp.float32)
    f1 = _edgefeat(feature1, p["Wf1a"], p["Wf1b"])
    agg1 = _sc_conv(xp, f1, src, dst, zeros)[:N]
    f2 = _edgefeat(feature2, p["Wf2a"], p["Wf2b"])
    agg2 = _sc_conv(xp, f2, src, dst, zeros)[:N]
    hpre, gsum, gcnt = _b1(agg1, agg2, xp, bid_row, p)
    vsum = _b2(hpre, bid_row, bid_col, gsum, gcnt, p["norm_ms"].reshape(1, H))
    return _b3(hpre, bid_col, gsum, gcnt, vsum, p)


# speedup vs baseline: 1.5697x; 1.5697x over previous
import functools

import jax
import jax.numpy as jnp
from jax import lax
from jax.experimental import pallas as pl
from jax.experimental.pallas import tpu as pltpu
from jax.experimental.pallas import tpu_sc as plsc

N = 10000
E = 320000
H = 128
NG = 64
F1 = 54
F2 = 18
MID = 64

NB = 2000
EB = 2000
NPAD = 10240
ROWS_PER_TILE = NPAD // 16
EDGES_PER_TILE = E // 16
CHUNK = 40
NCHUNK = EDGES_PER_TILE // CHUNK

_P = jax.lax.Precision.HIGHEST


def _swish(v):
    return v * jax.nn.sigmoid(v)


def _dot(a, b):
    return jnp.dot(a, b, precision=_P, preferred_element_type=jnp.float32)


def _xprime_body(x_ref, w_ref, b_ref, o_ref):
    o_ref[...] = _swish(_dot(x_ref[...], w_ref[...]) + b_ref[...])


def _xprime(x, W, b):
    return pl.pallas_call(
        _xprime_body,
        grid=(N // NB,),
        in_specs=[
            pl.BlockSpec((NB, H), lambda i: (i, 0)),
            pl.BlockSpec((H, H), lambda i: (0, 0)),
            pl.BlockSpec((1, H), lambda i: (0, 0)),
        ],
        out_specs=pl.BlockSpec((NB, H), lambda i: (i, 0)),
        out_shape=jax.ShapeDtypeStruct((N, H), jnp.float32),
    )(x, W, b)


def _edgefeat_body(f1_ref, f2_ref, wa1_ref, wb1_ref, wa2_ref, wb2_ref, o_ref):
    w1 = _dot(wa1_ref[...], wb1_ref[...])
    w2 = _dot(wa2_ref[...], wb2_ref[...])
    o_ref[0] = _dot(f1_ref[...], w1)
    o_ref[1] = _dot(f2_ref[...], w2)


def _edgefeat(feature1, feature2, Wf1a, Wf1b, Wf2a, Wf2b):
    return pl.pallas_call(
        _edgefeat_body,
        grid=(E // EB,),
        in_specs=[
            pl.BlockSpec((EB, F1), lambda i: (i, 0)),
            pl.BlockSpec((EB, F2), lambda i: (i, 0)),
            pl.BlockSpec((F1, MID), lambda i: (0, 0)),
            pl.BlockSpec((MID, H), lambda i: (0, 0)),
            pl.BlockSpec((F2, MID), lambda i: (0, 0)),
            pl.BlockSpec((MID, H), lambda i: (0, 0)),
        ],
        out_specs=pl.BlockSpec((2, EB, H), lambda i: (0, i, 0)),
        out_shape=jax.ShapeDtypeStruct((2, E, H), jnp.float32),
    )(feature1, feature2, Wf1a, Wf1b, Wf2a, Wf2b)


def _sc_agg(xp, f, src3, dst3, zeros):
    mesh = plsc.VectorSubcoreMesh(core_axis_name="c", subcore_axis_name="s")

    @functools.partial(
        pl.kernel,
        mesh=mesh,
        out_type=jax.ShapeDtypeStruct((2, NPAD, H), jnp.float32),
        scratch_types=[
            pltpu.VMEM((CHUNK,), jnp.int32),
            pltpu.VMEM((CHUNK,), jnp.int32),
            pltpu.VMEM((CHUNK,), jnp.int32),
            pltpu.VMEM((CHUNK,), jnp.int32),
            pltpu.VMEM((CHUNK, H), jnp.float32),
            pltpu.VMEM((CHUNK, H), jnp.float32),
            pltpu.VMEM((CHUNK, H), jnp.float32),
            pltpu.VMEM((CHUNK, H), jnp.float32),
            pltpu.VMEM((CHUNK, H), jnp.float32),
            pltpu.VMEM((CHUNK, H), jnp.float32),
            pltpu.VMEM_SHARED((NPAD, H), jnp.float32),
            pltpu.SemaphoreType.DMA,
            pltpu.SemaphoreType.DMA,
            pltpu.SemaphoreType.DMA,
            pltpu.SemaphoreType.DMA,
            pltpu.SemaphoreType.DMA,
            pltpu.SemaphoreType.DMA,
            pltpu.SemaphoreType.DMA,
            pltpu.SemaphoreType.DMA,
        ],
    )
    def k(x_hbm, f_hbm, src_hbm, dst_hbm, z_hbm, agg_hbm,
          sv0, sv1, dv0, dv1, xr0, xr1, fb0, fb1, pr0, pr1, agg_sh,
          sG0, sG1, sF0, sF1, sSI0, sSI1, sDI0, sDI1):
        c = lax.axis_index("c")
        s = lax.axis_index("s")
        sv = (sv0, sv1)
        dv = (dv0, dv1)
        xr = (xr0, xr1)
        fb = (fb0, fb1)
        pr = (pr0, pr1)
        sG = (sG0, sG1)
        sF = (sF0, sF1)
        sSI = (sSI0, sSI1)
        sDI = (sDI0, sDI1)
        nbase = s * ROWS_PER_TILE
        ebase = s * EDGES_PER_TILE

        def start_data(j, b):
            pltpu.async_copy(x_hbm.at[sv[b]], xr[b], sG[b])
            pltpu.async_copy(f_hbm.at[c, pl.ds(ebase + j * CHUNK, CHUNK)],
                             fb[b], sF[b])

        for b in range(2):
            pltpu.sync_copy(src_hbm.at[s, b], sv[b])
            pltpu.sync_copy(dst_hbm.at[s, b], dv[b])
            start_data(b, b)

        pltpu.sync_copy(z_hbm.at[pl.ds(nbase, ROWS_PER_TILE)],
                        agg_sh.at[pl.ds(nbase, ROWS_PER_TILE)])
        plsc.subcore_barrier()

        def pair(g, carry):
            for b in range(2):
                j = g * 2 + b
                pltpu.make_async_copy(x_hbm.at[sv[b]], xr[b], sG[b]).wait()

                @pl.when(g < NCHUNK // 2 - 1)
                def _():
                    pltpu.async_copy(src_hbm.at[s, j + 2], sv[b], sSI[b])

                pltpu.make_async_copy(
                    f_hbm.at[c, pl.ds(ebase + j * CHUNK, CHUNK)], fb[b],
                    sF[b]).wait()

                @pl.when(g >= 1)
                def _():
                    pltpu.make_async_copy(dst_hbm.at[s, j], dv[b], sDI[b]).wait()

                @plsc.parallel_loop(0, CHUNK, unroll=4)
                def _(r):
                    for kk in range(H // 16):
                        sl = pl.ds(kk * 16, 16)
                        pr[b][r, sl] = fb[b][r, sl] * xr[b][r, sl]

                pltpu.sync_copy(pr[b], agg_sh.at[dv[b]], add=True)

                @pl.when(g < NCHUNK // 2 - 1)
                def _():
                    pltpu.async_copy(dst_hbm.at[s, j + 2], dv[b], sDI[b])
                    pltpu.make_async_copy(src_hbm.at[s, j + 2], sv[b],
                                          sSI[b]).wait()
                    start_data(j + 2, b)
            return carry

        lax.fori_loop(0, NCHUNK // 2, pair, 0)
        plsc.subcore_barrier()
        pltpu.sync_copy(agg_sh.at[pl.ds(nbase, ROWS_PER_TILE)],
                        agg_hbm.at[c, pl.ds(nbase, ROWS_PER_TILE)])

    return k(xp, f, src3, dst3, zeros)


def _b1_body(agg1_ref, agg2_ref, x_ref, bid_ref,
             wrel1_ref, brel1_ref, wroot1_ref, w1_ref, b1_ref,
             wrel2_ref, brel2_ref, wroot2_ref, w2_ref, b2_ref,
             wcat_ref, bcat_ref, wl0_ref, bl0_ref, wl1_ref, bl1_ref,
             hpre_ref, gsum_ref, gsum2_ref, gcnt_ref):
    xb = x_ref[...]
    h1 = _dot(agg1_ref[...], wrel1_ref[...]) + brel1_ref[...] + _dot(xb, wroot1_ref[...])
    h1 = _swish(_dot(h1, w1_ref[...]) + b1_ref[...])
    h2 = _dot(agg2_ref[...], wrel2_ref[...]) + brel2_ref[...] + _dot(xb, wroot2_ref[...])
    h2 = _swish(_dot(h2, w2_ref[...]) + b2_ref[...])
    h = _dot(h1, wcat_ref[...][:H]) + _dot(h2, wcat_ref[...][H:]) + bcat_ref[...] + xb
    h = _swish(_dot(h, wl0_ref[...]) + bl0_ref[...]) + h
    h = _swish(_dot(h, wl1_ref[...]) + bl1_ref[...]) + h
    hpre_ref[...] = h
    ids = bid_ref[0]
    oh = (lax.broadcasted_iota(jnp.int32, (NG, NB), 0) == ids).astype(jnp.float32)
    psum = _dot(oh, h)
    psum2 = _dot(oh, h * h)
    pcnt = jnp.broadcast_to(jnp.sum(oh, axis=1, keepdims=True), (NG, H))

    @pl.when(pl.program_id(0) == 0)
    def _():
        gsum_ref[...] = psum
        gsum2_ref[...] = psum2
        gcnt_ref[...] = pcnt

    @pl.when(pl.program_id(0) != 0)
    def _():
        gsum_ref[...] += psum
        gsum2_ref[...] += psum2
        gcnt_ref[...] += pcnt


def _b1(agg1, agg2, xp, bid_row, p):
    wspec = pl.BlockSpec((H, H), lambda i: (0, 0))
    bspec = pl.BlockSpec((1, H), lambda i: (0, 0))
    return pl.pallas_call(
        _b1_body,
        grid=(N // NB,),
        in_specs=[
            pl.BlockSpec((NB, H), lambda i: (i, 0)),
            pl.BlockSpec((NB, H), lambda i: (i, 0)),
            pl.BlockSpec((NB, H), lambda i: (i, 0)),
            pl.BlockSpec((1, 1, NB), lambda i: (i, 0, 0)),
            wspec, bspec, wspec, wspec, bspec,
            wspec, bspec, wspec, wspec, bspec,
            pl.BlockSpec((2 * H, H), lambda i: (0, 0)), bspec,
            wspec, bspec, wspec, bspec,
        ],
        out_specs=[
            pl.BlockSpec((NB, H), lambda i: (i, 0)),
            pl.BlockSpec((NG, H), lambda i: (0, 0)),
            pl.BlockSpec((NG, H), lambda i: (0, 0)),
            pl.BlockSpec((NG, H), lambda i: (0, 0)),
        ],
        out_shape=[
            jax.ShapeDtypeStruct((N, H), jnp.float32),
            jax.ShapeDtypeStruct((NG, H), jnp.float32),
            jax.ShapeDtypeStruct((NG, H), jnp.float32),
            jax.ShapeDtypeStruct((NG, H), jnp.float32),
        ],
    )(agg1, agg2, xp, bid_row,
      p["Wrel1"], p["brel1"].reshape(1, H), p["Wroot1"], p["W1"], p["b1"].reshape(1, H),
      p["Wrel2"], p["brel2"].reshape(1, H), p["Wroot2"], p["W2"], p["b2"].reshape(1, H),
      p["Wcat"], p["bcat"].reshape(1, H), p["Wl0"], p["bl0"].reshape(1, H),
      p["Wl1"], p["bl1"].reshape(1, H))


def _b3_body(h_ref, bidc_ref, gsum_ref, gsum2_ref, gcnt_ref,
             nw_ref, nb_ref, ms_ref, wfin_ref, bfin_ref, o_ref):
    cnt = jnp.maximum(gcnt_ref[...], 1.0)
    mean = gsum_ref[...] / cnt
    meansq = gsum2_ref[...] / cnt
    ms = ms_ref[...]
    var = meansq - mean * mean * ms * (2.0 - ms)
    std = jnp.sqrt(var + 1e-5)
    idc = bidc_ref[...]
    ohc = (lax.broadcasted_iota(jnp.int32, (NB, NG), 1) == idc).astype(jnp.float32)
    cen = h_ref[...] - _dot(ohc, mean) * ms
    hn = nw_ref[...] * cen / _dot(ohc, std) + nb_ref[...]
    o_ref[...] = _dot(hn, wfin_ref[...]) + bfin_ref[...]


def _b3(hpre, bid_col, gsum, gsum2, gcnt, p):
    return pl.pallas_call(
        _b3_body,
        grid=(N // NB,),
        in_specs=[
            pl.BlockSpec((NB, H), lambda i: (i, 0)),
            pl.BlockSpec((NB, 1), lambda i: (i, 0)),
            pl.BlockSpec((NG, H), lambda i: (0, 0)),
            pl.BlockSpec((NG, H), lambda i: (0, 0)),
            pl.BlockSpec((NG, H), lambda i: (0, 0)),
            pl.BlockSpec((1, H), lambda i: (0, 0)),
            pl.BlockSpec((1, H), lambda i: (0, 0)),
            pl.BlockSpec((1, H), lambda i: (0, 0)),
            pl.BlockSpec((H, H), lambda i: (0, 0)),
            pl.BlockSpec((1, H), lambda i: (0, 0)),
        ],
        out_specs=pl.BlockSpec((NB, H), lambda i: (i, 0)),
        out_shape=jax.ShapeDtypeStruct((N, H), jnp.float32),
    )(hpre, bid_col, gsum, gsum2, gcnt,
      p["norm_w"].reshape(1, H), p["norm_b"].reshape(1, H),
      p["norm_ms"].reshape(1, H), p["Wfin"], p["bfin"].reshape(1, H))


def kernel(x, feature1, feature2, edge_index, batch, params):
    p = params
    ei = edge_index.astype(jnp.int32)
    src = ei[0].reshape(16, NCHUNK, CHUNK)
    dst = ei[1].reshape(16, NCHUNK, CHUNK)
    bid = batch.astype(jnp.int32)
    bid_row = bid.reshape(N // NB, 1, NB)
    bid_col = bid.reshape(N, 1)

    xp = _xprime(x, p["W_lin"], p["b_lin"].reshape(1, H))
    zeros = jnp.zeros((NPAD, H), jnp.float32)
    f = _edgefeat(feature1, feature2, p["Wf1a"], p["Wf1b"], p["Wf2a"], p["Wf2b"])
    agg = _sc_agg(xp, f, src, dst, zeros)
    hpre, gsum, gsum2, gcnt = _b1(agg[0, :N], agg[1, :N], xp, bid_row, p)
    return _b3(hpre, bid_col, gsum, gsum2, gcnt, p)


# speedup vs baseline: 1.9595x; 1.2483x over previous
import functools

import jax
import jax.numpy as jnp
from jax import lax
from jax.experimental import pallas as pl
from jax.experimental.pallas import tpu as pltpu
from jax.experimental.pallas import tpu_sc as plsc

N = 10000
E = 320000
H = 128
NG = 64
F1 = 54
F2 = 18
MID = 64

NB = 2000
EB = 3200
NPAD = 10240
ROWS_PER_TILE = NPAD // 16
EDGES_PER_TILE = E // 16
CHUNK = 40
NCHUNK = EDGES_PER_TILE // CHUNK

_P = jax.lax.Precision.HIGHEST


def _swish(v):
    return v * jax.nn.sigmoid(v)


def _dot(a, b):
    return jnp.dot(a, b, precision=_P, preferred_element_type=jnp.float32)


def _xprime_body(x_ref, w_ref, b_ref, o_ref):
    o_ref[...] = _swish(_dot(x_ref[...], w_ref[...]) + b_ref[...])


def _xprime(x, W, b):
    return pl.pallas_call(
        _xprime_body,
        grid=(N // NB,),
        in_specs=[
            pl.BlockSpec((NB, H), lambda i: (i, 0)),
            pl.BlockSpec((H, H), lambda i: (0, 0)),
            pl.BlockSpec((1, H), lambda i: (0, 0)),
        ],
        out_specs=pl.BlockSpec((NB, H), lambda i: (i, 0)),
        out_shape=jax.ShapeDtypeStruct((N, H), jnp.float32),
    )(x, W, b)


def _dotT(aT, b):
    return lax.dot_general(aT, b, (((0,), (0,)), ((), ())),
                           precision=_P, preferred_element_type=jnp.float32)


def _edgefeat_body(f1T_ref, f2T_ref, wa1_ref, wb1_ref, wa2_ref, wb2_ref, o_ref):
    w1 = _dot(wa1_ref[...], wb1_ref[...])
    w2 = _dot(wa2_ref[...], wb2_ref[...])
    o_ref[0] = _dotT(f1T_ref[...], w1)
    o_ref[1] = _dotT(f2T_ref[...], w2)


def _edgefeat(feature1T, feature2T, Wf1a, Wf1b, Wf2a, Wf2b):
    return pl.pallas_call(
        _edgefeat_body,
        grid=(E // EB,),
        in_specs=[
            pl.BlockSpec((F1, EB), lambda i: (0, i)),
            pl.BlockSpec((F2, EB), lambda i: (0, i)),
            pl.BlockSpec((F1, MID), lambda i: (0, 0)),
            pl.BlockSpec((MID, H), lambda i: (0, 0)),
            pl.BlockSpec((F2, MID), lambda i: (0, 0)),
            pl.BlockSpec((MID, H), lambda i: (0, 0)),
        ],
        out_specs=pl.BlockSpec((2, EB, H), lambda i: (0, i, 0)),
        out_shape=jax.ShapeDtypeStruct((2, E, H), jnp.float32),
    )(feature1T, feature2T, Wf1a, Wf1b, Wf2a, Wf2b)


def _sc_agg(xp, f, src3, dst3, zeros):
    mesh = plsc.VectorSubcoreMesh(core_axis_name="c", subcore_axis_name="s")

    @functools.partial(
        pl.kernel,
        mesh=mesh,
        out_type=jax.ShapeDtypeStruct((2, NPAD, H), jnp.float32),
        scratch_types=[
            pltpu.VMEM((CHUNK,), jnp.int32),
            pltpu.VMEM((CHUNK,), jnp.int32),
            pltpu.VMEM((CHUNK,), jnp.int32),
            pltpu.VMEM((CHUNK,), jnp.int32),
            pltpu.VMEM((CHUNK, H), jnp.float32),
            pltpu.VMEM((CHUNK, H), jnp.float32),
            pltpu.VMEM((CHUNK, H), jnp.float32),
            pltpu.VMEM((CHUNK, H), jnp.float32),
            pltpu.VMEM((CHUNK, H), jnp.float32),
            pltpu.VMEM((CHUNK, H), jnp.float32),
            pltpu.VMEM_SHARED((NPAD, H), jnp.float32),
            pltpu.SemaphoreType.DMA,
            pltpu.SemaphoreType.DMA,
            pltpu.SemaphoreType.DMA,
            pltpu.SemaphoreType.DMA,
            pltpu.SemaphoreType.DMA,
            pltpu.SemaphoreType.DMA,
            pltpu.SemaphoreType.DMA,
            pltpu.SemaphoreType.DMA,
        ],
    )
    def k(x_hbm, f_hbm, src_hbm, dst_hbm, z_hbm, agg_hbm,
          sv0, sv1, dv0, dv1, xr0, xr1, fb0, fb1, pr0, pr1, agg_sh,
          sG0, sG1, sF0, sF1, sSI0, sSI1, sDI0, sDI1):
        c = lax.axis_index("c")
        s = lax.axis_index("s")
        sv = (sv0, sv1)
        dv = (dv0, dv1)
        xr = (xr0, xr1)
        fb = (fb0, fb1)
        pr = (pr0, pr1)
        sG = (sG0, sG1)
        sF = (sF0, sF1)
        sSI = (sSI0, sSI1)
        sDI = (sDI0, sDI1)
        nbase = s * ROWS_PER_TILE
        ebase = s * EDGES_PER_TILE

        def start_data(j, b):
            pltpu.async_copy(x_hbm.at[sv[b]], xr[b], sG[b])
            pltpu.async_copy(f_hbm.at[c, pl.ds(ebase + j * CHUNK, CHUNK)],
                             fb[b], sF[b])

        for b in range(2):
            pltpu.sync_copy(src_hbm.at[s, b], sv[b])
            pltpu.sync_copy(dst_hbm.at[s, b], dv[b])
            start_data(b, b)

        pltpu.sync_copy(z_hbm.at[pl.ds(nbase, ROWS_PER_TILE)],
                        agg_sh.at[pl.ds(nbase, ROWS_PER_TILE)])
        plsc.subcore_barrier()

        def pair(g, carry):
            for b in range(2):
                j = g * 2 + b
                pltpu.make_async_copy(x_hbm.at[sv[b]], xr[b], sG[b]).wait()

                @pl.when(g < NCHUNK // 2 - 1)
                def _():
                    pltpu.async_copy(src_hbm.at[s, j + 2], sv[b], sSI[b])

                pltpu.make_async_copy(
                    f_hbm.at[c, pl.ds(ebase + j * CHUNK, CHUNK)], fb[b],
                    sF[b]).wait()

                @pl.when(g >= 1)
                def _():
                    pltpu.make_async_copy(dst_hbm.at[s, j], dv[b], sDI[b]).wait()

                @plsc.parallel_loop(0, CHUNK, unroll=4)
                def _(r):
                    for kk in range(H // 16):
                        sl = pl.ds(kk * 16, 16)
                        pr[b][r, sl] = fb[b][r, sl] * xr[b][r, sl]

                pltpu.sync_copy(pr[b], agg_sh.at[dv[b]], add=True)

                @pl.when(g < NCHUNK // 2 - 1)
                def _():
                    pltpu.async_copy(dst_hbm.at[s, j + 2], dv[b], sDI[b])
                    pltpu.make_async_copy(src_hbm.at[s, j + 2], sv[b],
                                          sSI[b]).wait()
                    start_data(j + 2, b)
            return carry

        lax.fori_loop(0, NCHUNK // 2, pair, 0)
        plsc.subcore_barrier()
        pltpu.sync_copy(agg_sh.at[pl.ds(nbase, ROWS_PER_TILE)],
                        agg_hbm.at[c, pl.ds(nbase, ROWS_PER_TILE)])

    return k(xp, f, src3, dst3, zeros)


def _b1_body(agg1_ref, agg2_ref, x_ref, bid_ref,
             wrel1_ref, brel1_ref, wroot1_ref, w1_ref, b1_ref,
             wrel2_ref, brel2_ref, wroot2_ref, w2_ref, b2_ref,
             wcat_ref, bcat_ref, wl0_ref, bl0_ref, wl1_ref, bl1_ref,
             hpre_ref, gsum_ref, gsum2_ref, gcnt_ref):
    xb = x_ref[...]
    h1 = _dot(agg1_ref[...], wrel1_ref[...]) + brel1_ref[...] + _dot(xb, wroot1_ref[...])
    h1 = _swish(_dot(h1, w1_ref[...]) + b1_ref[...])
    h2 = _dot(agg2_ref[...], wrel2_ref[...]) + brel2_ref[...] + _dot(xb, wroot2_ref[...])
    h2 = _swish(_dot(h2, w2_ref[...]) + b2_ref[...])
    h = _dot(h1, wcat_ref[...][:H]) + _dot(h2, wcat_ref[...][H:]) + bcat_ref[...] + xb
    h = _swish(_dot(h, wl0_ref[...]) + bl0_ref[...]) + h
    h = _swish(_dot(h, wl1_ref[...]) + bl1_ref[...]) + h
    hpre_ref[...] = h
    ids = bid_ref[0]
    oh = (lax.broadcasted_iota(jnp.int32, (NG, NB), 0) == ids).astype(jnp.float32)
    psum = _dot(oh, h)
    psum2 = _dot(oh, h * h)
    pcnt = jnp.broadcast_to(jnp.sum(oh, axis=1, keepdims=True), (NG, H))

    @pl.when(pl.program_id(0) == 0)
    def _():
        gsum_ref[...] = psum
        gsum2_ref[...] = psum2
        gcnt_ref[...] = pcnt

    @pl.when(pl.program_id(0) != 0)
    def _():
        gsum_ref[...] += psum
        gsum2_ref[...] += psum2
        gcnt_ref[...] += pcnt


def _b1(agg1, agg2, xp, bid_row, p):
    wspec = pl.BlockSpec((H, H), lambda i: (0, 0))
    bspec = pl.BlockSpec((1, H), lambda i: (0, 0))
    return pl.pallas_call(
        _b1_body,
        grid=(N // NB,),
        in_specs=[
            pl.BlockSpec((NB, H), lambda i: (i, 0)),
            pl.BlockSpec((NB, H), lambda i: (i, 0)),
            pl.BlockSpec((NB, H), lambda i: (i, 0)),
            pl.BlockSpec((1, 1, NB), lambda i: (i, 0, 0)),
            wspec, bspec, wspec, wspec, bspec,
            wspec, bspec, wspec, wspec, bspec,
            pl.BlockSpec((2 * H, H), lambda i: (0, 0)), bspec,
            wspec, bspec, wspec, bspec,
        ],
        out_specs=[
            pl.BlockSpec((NB, H), lambda i: (i, 0)),
            pl.BlockSpec((NG, H), lambda i: (0, 0)),
            pl.BlockSpec((NG, H), lambda i: (0, 0)),
            pl.BlockSpec((NG, H), lambda i: (0, 0)),
        ],
        out_shape=[
            jax.ShapeDtypeStruct((N, H), jnp.float32),
            jax.ShapeDtypeStruct((NG, H), jnp.float32),
            jax.ShapeDtypeStruct((NG, H), jnp.float32),
            jax.ShapeDtypeStruct((NG, H), jnp.float32),
        ],
    )(agg1, agg2, xp, bid_row,
      p["Wrel1"], p["brel1"].reshape(1, H), p["Wroot1"], p["W1"], p["b1"].reshape(1, H),
      p["Wrel2"], p["brel2"].reshape(1, H), p["Wroot2"], p["W2"], p["b2"].reshape(1, H),
      p["Wcat"], p["bcat"].reshape(1, H), p["Wl0"], p["bl0"].reshape(1, H),
      p["Wl1"], p["bl1"].reshape(1, H))


def _b3_body(h_ref, bidc_ref, gsum_ref, gsum2_ref, gcnt_ref,
             nw_ref, nb_ref, ms_ref, wfin_ref, bfin_ref, o_ref):
    cnt = jnp.maximum(gcnt_ref[...], 1.0)
    mean = gsum_ref[...] / cnt
    meansq = gsum2_ref[...] / cnt
    ms = ms_ref[...]
    var = meansq - mean * mean * ms * (2.0 - ms)
    std = jnp.sqrt(var + 1e-5)
    idc = bidc_ref[...]
    ohc = (lax.broadcasted_iota(jnp.int32, (NB, NG), 1) == idc).astype(jnp.float32)
    cen = h_ref[...] - _dot(ohc, mean) * ms
    hn = nw_ref[...] * cen / _dot(ohc, std) + nb_ref[...]
    o_ref[...] = _dot(hn, wfin_ref[...]) + bfin_ref[...]


def _b3(hpre, bid_col, gsum, gsum2, gcnt, p):
    return pl.pallas_call(
        _b3_body,
        grid=(N // NB,),
        in_specs=[
            pl.BlockSpec((NB, H), lambda i: (i, 0)),
            pl.BlockSpec((NB, 1), lambda i: (i, 0)),
            pl.BlockSpec((NG, H), lambda i: (0, 0)),
            pl.BlockSpec((NG, H), lambda i: (0, 0)),
            pl.BlockSpec((NG, H), lambda i: (0, 0)),
            pl.BlockSpec((1, H), lambda i: (0, 0)),
            pl.BlockSpec((1, H), lambda i: (0, 0)),
            pl.BlockSpec((1, H), lambda i: (0, 0)),
            pl.BlockSpec((H, H), lambda i: (0, 0)),
            pl.BlockSpec((1, H), lambda i: (0, 0)),
        ],
        out_specs=pl.BlockSpec((NB, H), lambda i: (i, 0)),
        out_shape=jax.ShapeDtypeStruct((N, H), jnp.float32),
    )(hpre, bid_col, gsum, gsum2, gcnt,
      p["norm_w"].reshape(1, H), p["norm_b"].reshape(1, H),
      p["norm_ms"].reshape(1, H), p["Wfin"], p["bfin"].reshape(1, H))


def kernel(x, feature1, feature2, edge_index, batch, params):
    p = params
    ei = edge_index.astype(jnp.int32)
    src = ei[0].reshape(16, NCHUNK, CHUNK)
    dst = ei[1].reshape(16, NCHUNK, CHUNK)
    bid = batch.astype(jnp.int32)
    bid_row = bid.reshape(N // NB, 1, NB)
    bid_col = bid.reshape(N, 1)

    xp = _xprime(x, p["W_lin"], p["b_lin"].reshape(1, H))
    zeros = jnp.zeros((NPAD, H), jnp.float32)
    f = _edgefeat(feature1.T, feature2.T, p["Wf1a"], p["Wf1b"], p["Wf2a"], p["Wf2b"])
    agg = _sc_agg(xp, f, src, dst, zeros)
    hpre, gsum, gsum2, gcnt = _b1(agg[0, :N], agg[1, :N], xp, bid_row, p)
    return _b3(hpre, bid_col, gsum, gsum2, gcnt, p)


# speedup vs baseline: 2.2251x; 1.1355x over previous
import functools

import jax
import jax.numpy as jnp
from jax import lax
from jax.experimental import pallas as pl
from jax.experimental.pallas import tpu as pltpu
from jax.experimental.pallas import tpu_sc as plsc

N = 10000
E = 320000
H = 128
NG = 64
F1 = 54
F2 = 18
MID = 64

NB = 2000
EB = 3200
NPAD = 10240
ROWS_PER_TILE = NPAD // 16
EHALF = E // 2
EDGES_PER_TILE = EHALF // 16
CHUNK = 40
NCHUNK = EDGES_PER_TILE // CHUNK

_P = jax.lax.Precision.HIGHEST


def _swish(v):
    return v * jax.nn.sigmoid(v)


def _dot(a, b):
    return jnp.dot(a, b, precision=_P, preferred_element_type=jnp.float32)


def _xprime_body(x_ref, w_ref, b_ref, o_ref):
    o_ref[...] = _swish(_dot(x_ref[...], w_ref[...]) + b_ref[...])


def _xprime(x, W, b):
    return pl.pallas_call(
        _xprime_body,
        grid=(N // NB,),
        in_specs=[
            pl.BlockSpec((NB, H), lambda i: (i, 0)),
            pl.BlockSpec((H, H), lambda i: (0, 0)),
            pl.BlockSpec((1, H), lambda i: (0, 0)),
        ],
        out_specs=pl.BlockSpec((NB, H), lambda i: (i, 0)),
        out_shape=jax.ShapeDtypeStruct((N, H), jnp.float32),
    )(x, W, b)


def _dotT(aT, b):
    return lax.dot_general(aT, b, (((0,), (0,)), ((), ())),
                           precision=_P, preferred_element_type=jnp.float32)


def _edgefeat_body(f1T_ref, f2T_ref, wa1_ref, wb1_ref, wa2_ref, wb2_ref, o_ref):
    w1 = _dot(wa1_ref[...], wb1_ref[...])
    w2 = _dot(wa2_ref[...], wb2_ref[...])
    o_ref[0] = _dotT(f1T_ref[...], w1)
    o_ref[1] = _dotT(f2T_ref[...], w2)


def _edgefeat(half, feature1T, feature2T, Wf1a, Wf1b, Wf2a, Wf2b):
    off = half * (EHALF // EB)
    return pl.pallas_call(
        _edgefeat_body,
        grid=(EHALF // EB,),
        in_specs=[
            pl.BlockSpec((F1, EB), lambda i: (0, i + off)),
            pl.BlockSpec((F2, EB), lambda i: (0, i + off)),
            pl.BlockSpec((F1, MID), lambda i: (0, 0)),
            pl.BlockSpec((MID, H), lambda i: (0, 0)),
            pl.BlockSpec((F2, MID), lambda i: (0, 0)),
            pl.BlockSpec((MID, H), lambda i: (0, 0)),
        ],
        out_specs=pl.BlockSpec((2, EB, H), lambda i: (0, i, 0)),
        out_shape=jax.ShapeDtypeStruct((2, EHALF, H), jnp.float32),
    )(feature1T, feature2T, Wf1a, Wf1b, Wf2a, Wf2b)


def _sc_agg(half, xp, f, src4, dst4, zeros):
    mesh = plsc.VectorSubcoreMesh(core_axis_name="c", subcore_axis_name="s")

    @functools.partial(
        pl.kernel,
        mesh=mesh,
        out_type=jax.ShapeDtypeStruct((2, NPAD, H), jnp.float32),
        scratch_types=[
            pltpu.VMEM((CHUNK,), jnp.int32),
            pltpu.VMEM((CHUNK,), jnp.int32),
            pltpu.VMEM((CHUNK,), jnp.int32),
            pltpu.VMEM((CHUNK,), jnp.int32),
            pltpu.VMEM((CHUNK, H), jnp.float32),
            pltpu.VMEM((CHUNK, H), jnp.float32),
            pltpu.VMEM((CHUNK, H), jnp.float32),
            pltpu.VMEM((CHUNK, H), jnp.float32),
            pltpu.VMEM((CHUNK, H), jnp.float32),
            pltpu.VMEM((CHUNK, H), jnp.float32),
            pltpu.VMEM_SHARED((NPAD, H), jnp.float32),
            pltpu.SemaphoreType.DMA,
            pltpu.SemaphoreType.DMA,
            pltpu.SemaphoreType.DMA,
            pltpu.SemaphoreType.DMA,
            pltpu.SemaphoreType.DMA,
            pltpu.SemaphoreType.DMA,
            pltpu.SemaphoreType.DMA,
            pltpu.SemaphoreType.DMA,
        ],
    )
    def k(x_hbm, f_hbm, src_hbm, dst_hbm, z_hbm, agg_hbm,
          sv0, sv1, dv0, dv1, xr0, xr1, fb0, fb1, pr0, pr1, agg_sh,
          sG0, sG1, sF0, sF1, sSI0, sSI1, sDI0, sDI1):
        c = lax.axis_index("c")
        s = lax.axis_index("s")
        sv = (sv0, sv1)
        dv = (dv0, dv1)
        xr = (xr0, xr1)
        fb = (fb0, fb1)
        pr = (pr0, pr1)
        sG = (sG0, sG1)
        sF = (sF0, sF1)
        sSI = (sSI0, sSI1)
        sDI = (sDI0, sDI1)
        nbase = s * ROWS_PER_TILE
        ebase = s * EDGES_PER_TILE

        def start_data(j, b):
            pltpu.async_copy(x_hbm.at[sv[b]], xr[b], sG[b])
            pltpu.async_copy(f_hbm.at[c, pl.ds(ebase + j * CHUNK, CHUNK)],
                             fb[b], sF[b])

        for b in range(2):
            pltpu.sync_copy(src_hbm.at[half, s, b], sv[b])
            pltpu.sync_copy(dst_hbm.at[half, s, b], dv[b])
            start_data(b, b)

        pltpu.sync_copy(z_hbm.at[pl.ds(nbase, ROWS_PER_TILE)],
                        agg_sh.at[pl.ds(nbase, ROWS_PER_TILE)])
        plsc.subcore_barrier()

        def pair(g, carry):
            for b in range(2):
                j = g * 2 + b
                pltpu.make_async_copy(x_hbm.at[sv[b]], xr[b], sG[b]).wait()

                @pl.when(g < NCHUNK // 2 - 1)
                def _():
                    pltpu.async_copy(src_hbm.at[half, s, j + 2], sv[b], sSI[b])

                pltpu.make_async_copy(
                    f_hbm.at[c, pl.ds(ebase + j * CHUNK, CHUNK)], fb[b],
                    sF[b]).wait()

                @pl.when(g >= 1)
                def _():
                    pltpu.make_async_copy(dst_hbm.at[half, s, j], dv[b],
                                          sDI[b]).wait()

                @plsc.parallel_loop(0, CHUNK, unroll=4)
                def _(r):
                    for kk in range(H // 16):
                        sl = pl.ds(kk * 16, 16)
                        pr[b][r, sl] = fb[b][r, sl] * xr[b][r, sl]

                pltpu.sync_copy(pr[b], agg_sh.at[dv[b]], add=True)

                @pl.when(g < NCHUNK // 2 - 1)
                def _():
                    pltpu.async_copy(dst_hbm.at[half, s, j + 2], dv[b], sDI[b])
                    pltpu.make_async_copy(src_hbm.at[half, s, j + 2], sv[b],
                                          sSI[b]).wait()
                    start_data(j + 2, b)
            return carry

        lax.fori_loop(0, NCHUNK // 2, pair, 0)
        plsc.subcore_barrier()
        pltpu.sync_copy(agg_sh.at[pl.ds(nbase, ROWS_PER_TILE)],
                        agg_hbm.at[c, pl.ds(nbase, ROWS_PER_TILE)])

    return k(xp, f, src4, dst4, zeros)


def _b1_body(aggA_ref, aggB_ref, x_ref, bid_ref,
             wrel1_ref, brel1_ref, wroot1_ref, w1_ref, b1_ref,
             wrel2_ref, brel2_ref, wroot2_ref, w2_ref, b2_ref,
             wcat_ref, bcat_ref, wl0_ref, bl0_ref, wl1_ref, bl1_ref,
             hpre_ref, gsum_ref, gsum2_ref, gcnt_ref):
    xb = x_ref[...]
    agg1 = aggA_ref[0] + aggB_ref[0]
    agg2 = aggA_ref[1] + aggB_ref[1]
    h1 = _dot(agg1, wrel1_ref[...]) + brel1_ref[...] + _dot(xb, wroot1_ref[...])
    h1 = _swish(_dot(h1, w1_ref[...]) + b1_ref[...])
    h2 = _dot(agg2, wrel2_ref[...]) + brel2_ref[...] + _dot(xb, wroot2_ref[...])
    h2 = _swish(_dot(h2, w2_ref[...]) + b2_ref[...])
    h = _dot(h1, wcat_ref[...][:H]) + _dot(h2, wcat_ref[...][H:]) + bcat_ref[...] + xb
    h = _swish(_dot(h, wl0_ref[...]) + bl0_ref[...]) + h
    h = _swish(_dot(h, wl1_ref[...]) + bl1_ref[...]) + h
    hpre_ref[...] = h
    ids = bid_ref[0]
    oh = (lax.broadcasted_iota(jnp.int32, (NG, NB), 0) == ids).astype(jnp.float32)
    psum = _dot(oh, h)
    psum2 = _dot(oh, h * h)
    pcnt = jnp.broadcast_to(jnp.sum(oh, axis=1, keepdims=True), (NG, H))

    @pl.when(pl.program_id(0) == 0)
    def _():
        gsum_ref[...] = psum
        gsum2_ref[...] = psum2
        gcnt_ref[...] = pcnt

    @pl.when(pl.program_id(0) != 0)
    def _():
        gsum_ref[...] += psum
        gsum2_ref[...] += psum2
        gcnt_ref[...] += pcnt


def _b1(aggA, aggB, xp, bid_row, p):
    wspec = pl.BlockSpec((H, H), lambda i: (0, 0))
    bspec = pl.BlockSpec((1, H), lambda i: (0, 0))
    return pl.pallas_call(
        _b1_body,
        grid=(N // NB,),
        in_specs=[
            pl.BlockSpec((2, NB, H), lambda i: (0, i, 0)),
            pl.BlockSpec((2, NB, H), lambda i: (0, i, 0)),
            pl.BlockSpec((NB, H), lambda i: (i, 0)),
            pl.BlockSpec((1, 1, NB), lambda i: (i, 0, 0)),
            wspec, bspec, wspec, wspec, bspec,
            wspec, bspec, wspec, wspec, bspec,
            pl.BlockSpec((2 * H, H), lambda i: (0, 0)), bspec,
            wspec, bspec, wspec, bspec,
        ],
        out_specs=[
            pl.BlockSpec((NB, H), lambda i: (i, 0)),
            pl.BlockSpec((NG, H), lambda i: (0, 0)),
            pl.BlockSpec((NG, H), lambda i: (0, 0)),
            pl.BlockSpec((NG, H), lambda i: (0, 0)),
        ],
        out_shape=[
            jax.ShapeDtypeStruct((N, H), jnp.float32),
            jax.ShapeDtypeStruct((NG, H), jnp.float32),
            jax.ShapeDtypeStruct((NG, H), jnp.float32),
            jax.ShapeDtypeStruct((NG, H), jnp.float32),
        ],
    )(aggA, aggB, xp, bid_row,
      p["Wrel1"], p["brel1"].reshape(1, H), p["Wroot1"], p["W1"], p["b1"].reshape(1, H),
      p["Wrel2"], p["brel2"].reshape(1, H), p["Wroot2"], p["W2"], p["b2"].reshape(1, H),
      p["Wcat"], p["bcat"].reshape(1, H), p["Wl0"], p["bl0"].reshape(1, H),
      p["Wl1"], p["bl1"].reshape(1, H))


def _b3_body(h_ref, bidc_ref, gsum_ref, gsum2_ref, gcnt_ref,
             nw_ref, nb_ref, ms_ref, wfin_ref, bfin_ref, o_ref):
    cnt = jnp.maximum(gcnt_ref[...], 1.0)
    mean = gsum_ref[...] / cnt
    meansq = gsum2_ref[...] / cnt
    ms = ms_ref[...]
    var = meansq - mean * mean * ms * (2.0 - ms)
    std = jnp.sqrt(var + 1e-5)
    idc = bidc_ref[...]
    ohc = (lax.broadcasted_iota(jnp.int32, (NB, NG), 1) == idc).astype(jnp.float32)
    cen = h_ref[...] - _dot(ohc, mean) * ms
    hn = nw_ref[...] * cen / _dot(ohc, std) + nb_ref[...]
    o_ref[...] = _dot(hn, wfin_ref[...]) + bfin_ref[...]


def _b3(hpre, bid_col, gsum, gsum2, gcnt, p):
    return pl.pallas_call(
        _b3_body,
        grid=(N // NB,),
        in_specs=[
            pl.BlockSpec((NB, H), lambda i: (i, 0)),
            pl.BlockSpec((NB, 1), lambda i: (i, 0)),
            pl.BlockSpec((NG, H), lambda i: (0, 0)),
            pl.BlockSpec((NG, H), lambda i: (0, 0)),
            pl.BlockSpec((NG, H), lambda i: (0, 0)),
            pl.BlockSpec((1, H), lambda i: (0, 0)),
            pl.BlockSpec((1, H), lambda i: (0, 0)),
            pl.BlockSpec((1, H), lambda i: (0, 0)),
            pl.BlockSpec((H, H), lambda i: (0, 0)),
            pl.BlockSpec((1, H), lambda i: (0, 0)),
        ],
        out_specs=pl.BlockSpec((NB, H), lambda i: (i, 0)),
        out_shape=jax.ShapeDtypeStruct((N, H), jnp.float32),
    )(hpre, bid_col, gsum, gsum2, gcnt,
      p["norm_w"].reshape(1, H), p["norm_b"].reshape(1, H),
      p["norm_ms"].reshape(1, H), p["Wfin"], p["bfin"].reshape(1, H))


def kernel(x, feature1, feature2, edge_index, batch, params):
    p = params
    ei = edge_index.astype(jnp.int32)
    src = ei[0].reshape(2, 16, NCHUNK, CHUNK)
    dst = ei[1].reshape(2, 16, NCHUNK, CHUNK)
    bid = batch.astype(jnp.int32)
    bid_row = bid.reshape(N // NB, 1, NB)
    bid_col = bid.reshape(N, 1)

    xp = _xprime(x, p["W_lin"], p["b_lin"].reshape(1, H))
    zeros = jnp.zeros((NPAD, H), jnp.float32)
    f1T, f2T = feature1.T, feature2.T
    fA = _edgefeat(0, f1T, f2T, p["Wf1a"], p["Wf1b"], p["Wf2a"], p["Wf2b"])
    aggA = _sc_agg(0, xp, fA, src, dst, zeros)
    fB = _edgefeat(1, f1T, f2T, p["Wf1a"], p["Wf1b"], p["Wf2a"], p["Wf2b"])
    aggB = _sc_agg(1, xp, fB, src, dst, zeros)
    hpre, gsum, gsum2, gcnt = _b1(aggA[:, :N], aggB[:, :N], xp, bid_row, p)
    return _b3(hpre, bid_col, gsum, gsum2, gcnt, p)


# speedup vs baseline: 2.2540x; 1.0130x over previous
import functools

import jax
import jax.numpy as jnp
from jax import lax
from jax.experimental import pallas as pl
from jax.experimental.pallas import tpu as pltpu
from jax.experimental.pallas import tpu_sc as plsc

N = 10000
E = 320000
H = 128
NG = 64
F1 = 54
F2 = 18
MID = 64

NB = 2000
EB = 6400
NPAD = 10240
ROWS_PER_TILE = NPAD // 16
EHALF = E // 2
EDGES_PER_TILE = EHALF // 16
CHUNK = 40
NCHUNK = EDGES_PER_TILE // CHUNK

_P = jax.lax.Precision.HIGHEST


def _swish(v):
    return v * jax.nn.sigmoid(v)


def _dot(a, b):
    return jnp.dot(a, b, precision=_P, preferred_element_type=jnp.float32)


def _xprime_body(x_ref, w_ref, b_ref, o_ref):
    o_ref[...] = _swish(_dot(x_ref[...], w_ref[...]) + b_ref[...])


def _xprime(x, W, b):
    return pl.pallas_call(
        _xprime_body,
        grid=(N // NB,),
        in_specs=[
            pl.BlockSpec((NB, H), lambda i: (i, 0)),
            pl.BlockSpec((H, H), lambda i: (0, 0)),
            pl.BlockSpec((1, H), lambda i: (0, 0)),
        ],
        out_specs=pl.BlockSpec((NB, H), lambda i: (i, 0)),
        out_shape=jax.ShapeDtypeStruct((N, H), jnp.float32),
    )(x, W, b)


def _dotT(aT, b):
    return lax.dot_general(aT, b, (((0,), (0,)), ((), ())),
                           precision=_P, preferred_element_type=jnp.float32)


def _edgefeat_body(f1T_ref, f2T_ref, wa1_ref, wb1_ref, wa2_ref, wb2_ref, o_ref):
    w1 = _dot(wa1_ref[...], wb1_ref[...])
    w2 = _dot(wa2_ref[...], wb2_ref[...])
    o_ref[0] = _dotT(f1T_ref[...], w1)
    o_ref[1] = _dotT(f2T_ref[...], w2)


def _edgefeat(half, feature1T, feature2T, Wf1a, Wf1b, Wf2a, Wf2b):
    off = half * (EHALF // EB)
    return pl.pallas_call(
        _edgefeat_body,
        grid=(EHALF // EB,),
        in_specs=[
            pl.BlockSpec((F1, EB), lambda i: (0, i + off)),
            pl.BlockSpec((F2, EB), lambda i: (0, i + off)),
            pl.BlockSpec((F1, MID), lambda i: (0, 0)),
            pl.BlockSpec((MID, H), lambda i: (0, 0)),
            pl.BlockSpec((F2, MID), lambda i: (0, 0)),
            pl.BlockSpec((MID, H), lambda i: (0, 0)),
        ],
        out_specs=pl.BlockSpec((2, EB, H), lambda i: (0, i, 0)),
        out_shape=jax.ShapeDtypeStruct((2, EHALF, H), jnp.float32),
    )(feature1T, feature2T, Wf1a, Wf1b, Wf2a, Wf2b)


def _sc_agg(half, xp, f, src4, dst4, zeros):
    mesh = plsc.VectorSubcoreMesh(core_axis_name="c", subcore_axis_name="s")

    @functools.partial(
        pl.kernel,
        mesh=mesh,
        out_type=jax.ShapeDtypeStruct((2, NPAD, H), jnp.float32),
        scratch_types=[
            pltpu.VMEM((CHUNK,), jnp.int32),
            pltpu.VMEM((CHUNK,), jnp.int32),
            pltpu.VMEM((CHUNK,), jnp.int32),
            pltpu.VMEM((CHUNK,), jnp.int32),
            pltpu.VMEM((CHUNK, H), jnp.float32),
            pltpu.VMEM((CHUNK, H), jnp.float32),
            pltpu.VMEM((CHUNK, H), jnp.float32),
            pltpu.VMEM((CHUNK, H), jnp.float32),
            pltpu.VMEM((CHUNK, H), jnp.float32),
            pltpu.VMEM((CHUNK, H), jnp.float32),
            pltpu.VMEM_SHARED((NPAD, H), jnp.float32),
            pltpu.SemaphoreType.DMA,
            pltpu.SemaphoreType.DMA,
            pltpu.SemaphoreType.DMA,
            pltpu.SemaphoreType.DMA,
            pltpu.SemaphoreType.DMA,
            pltpu.SemaphoreType.DMA,
            pltpu.SemaphoreType.DMA,
            pltpu.SemaphoreType.DMA,
            pltpu.SemaphoreType.DMA,
            pltpu.SemaphoreType.DMA,
        ],
    )
    def k(x_hbm, f_hbm, src_hbm, dst_hbm, z_hbm, agg_hbm,
          sv0, sv1, dv0, dv1, xr0, xr1, fb0, fb1, pr0, pr1, agg_sh,
          sG0, sG1, sF0, sF1, sSI0, sSI1, sDI0, sDI1, sS0, sS1):
        c = lax.axis_index("c")
        s = lax.axis_index("s")
        sv = (sv0, sv1)
        dv = (dv0, dv1)
        xr = (xr0, xr1)
        fb = (fb0, fb1)
        pr = (pr0, pr1)
        sG = (sG0, sG1)
        sF = (sF0, sF1)
        sSI = (sSI0, sSI1)
        sDI = (sDI0, sDI1)
        sS = (sS0, sS1)
        nbase = s * ROWS_PER_TILE
        ebase = s * EDGES_PER_TILE

        def start_data(j, b):
            pltpu.async_copy(x_hbm.at[sv[b]], xr[b], sG[b])
            pltpu.async_copy(f_hbm.at[c, pl.ds(ebase + j * CHUNK, CHUNK)],
                             fb[b], sF[b])

        for b in range(2):
            pltpu.sync_copy(src_hbm.at[half, s, b], sv[b])
            pltpu.sync_copy(dst_hbm.at[half, s, b], dv[b])
            start_data(b, b)

        pltpu.sync_copy(z_hbm.at[pl.ds(nbase, ROWS_PER_TILE)],
                        agg_sh.at[pl.ds(nbase, ROWS_PER_TILE)])
        plsc.subcore_barrier()

        def pair(g, carry):
            for b in range(2):
                j = g * 2 + b
                pltpu.make_async_copy(x_hbm.at[sv[b]], xr[b], sG[b]).wait()

                @pl.when(g < NCHUNK // 2 - 1)
                def _():
                    pltpu.async_copy(src_hbm.at[half, s, j + 2], sv[b], sSI[b])

                pltpu.make_async_copy(
                    f_hbm.at[c, pl.ds(ebase + j * CHUNK, CHUNK)], fb[b],
                    sF[b]).wait()

                @plsc.parallel_loop(0, CHUNK, unroll=4)
                def _(r):
                    for kk in range(H // 16):
                        sl = pl.ds(kk * 16, 16)
                        pr[b][r, sl] = fb[b][r, sl] * xr[b][r, sl]

                if b == 0:
                    @pl.when(g >= 1)
                    def _():
                        pltpu.make_async_copy(pr[1 - b], agg_sh.at[dv[1 - b]],
                                              sS[1 - b]).wait()
                else:
                    pltpu.make_async_copy(pr[1 - b], agg_sh.at[dv[1 - b]],
                                          sS[1 - b]).wait()

                @pl.when(g >= 1)
                def _():
                    pltpu.make_async_copy(dst_hbm.at[half, s, j], dv[b],
                                          sDI[b]).wait()

                pltpu.async_copy(pr[b], agg_sh.at[dv[b]], sS[b], add=True)

                @pl.when((g >= 1) if b == 0 else (g < NCHUNK // 2 - 1))
                def _():
                    pltpu.async_copy(dst_hbm.at[half, s, j + 1], dv[1 - b],
                                     sDI[1 - b])

                @pl.when(g < NCHUNK // 2 - 1)
                def _():
                    pltpu.make_async_copy(src_hbm.at[half, s, j + 2], sv[b],
                                          sSI[b]).wait()
                    start_data(j + 2, b)
            return carry

        lax.fori_loop(0, NCHUNK // 2, pair, 0)
        pltpu.make_async_copy(pr[1], agg_sh.at[dv[1]], sS[1]).wait()
        plsc.subcore_barrier()
        pltpu.sync_copy(agg_sh.at[pl.ds(nbase, ROWS_PER_TILE)],
                        agg_hbm.at[c, pl.ds(nbase, ROWS_PER_TILE)])

    return k(xp, f, src4, dst4, zeros)


def _b1_body(aggA_ref, aggB_ref, x_ref, bid_ref,
             wrel1_ref, brel1_ref, wroot1_ref, w1_ref, b1_ref,
             wrel2_ref, brel2_ref, wroot2_ref, w2_ref, b2_ref,
             wcat_ref, bcat_ref, wl0_ref, bl0_ref, wl1_ref, bl1_ref,
             hpre_ref, gsum_ref, gsum2_ref, gcnt_ref):
    xb = x_ref[...]
    agg1 = aggA_ref[0] + aggB_ref[0]
    agg2 = aggA_ref[1] + aggB_ref[1]
    h1 = _dot(agg1, wrel1_ref[...]) + brel1_ref[...] + _dot(xb, wroot1_ref[...])
    h1 = _swish(_dot(h1, w1_ref[...]) + b1_ref[...])
    h2 = _dot(agg2, wrel2_ref[...]) + brel2_ref[...] + _dot(xb, wroot2_ref[...])
    h2 = _swish(_dot(h2, w2_ref[...]) + b2_ref[...])
    h = _dot(h1, wcat_ref[...][:H]) + _dot(h2, wcat_ref[...][H:]) + bcat_ref[...] + xb
    h = _swish(_dot(h, wl0_ref[...]) + bl0_ref[...]) + h
    h = _swish(_dot(h, wl1_ref[...]) + bl1_ref[...]) + h
    hpre_ref[...] = h
    ids = bid_ref[0]
    oh = (lax.broadcasted_iota(jnp.int32, (NG, NB), 0) == ids).astype(jnp.float32)
    psum = _dot(oh, h)
    psum2 = _dot(oh, h * h)
    pcnt = jnp.broadcast_to(jnp.sum(oh, axis=1, keepdims=True), (NG, H))

    @pl.when(pl.program_id(0) == 0)
    def _():
        gsum_ref[...] = psum
        gsum2_ref[...] = psum2
        gcnt_ref[...] = pcnt

    @pl.when(pl.program_id(0) != 0)
    def _():
        gsum_ref[...] += psum
        gsum2_ref[...] += psum2
        gcnt_ref[...] += pcnt


def _b1(aggA, aggB, xp, bid_row, p):
    wspec = pl.BlockSpec((H, H), lambda i: (0, 0))
    bspec = pl.BlockSpec((1, H), lambda i: (0, 0))
    return pl.pallas_call(
        _b1_body,
        grid=(N // NB,),
        in_specs=[
            pl.BlockSpec((2, NB, H), lambda i: (0, i, 0)),
            pl.BlockSpec((2, NB, H), lambda i: (0, i, 0)),
            pl.BlockSpec((NB, H), lambda i: (i, 0)),
            pl.BlockSpec((1, 1, NB), lambda i: (i, 0, 0)),
            wspec, bspec, wspec, wspec, bspec,
            wspec, bspec, wspec, wspec, bspec,
            pl.BlockSpec((2 * H, H), lambda i: (0, 0)), bspec,
            wspec, bspec, wspec, bspec,
        ],
        out_specs=[
            pl.BlockSpec((NB, H), lambda i: (i, 0)),
            pl.BlockSpec((NG, H), lambda i: (0, 0)),
            pl.BlockSpec((NG, H), lambda i: (0, 0)),
            pl.BlockSpec((NG, H), lambda i: (0, 0)),
        ],
        out_shape=[
            jax.ShapeDtypeStruct((N, H), jnp.float32),
            jax.ShapeDtypeStruct((NG, H), jnp.float32),
            jax.ShapeDtypeStruct((NG, H), jnp.float32),
            jax.ShapeDtypeStruct((NG, H), jnp.float32),
        ],
    )(aggA, aggB, xp, bid_row,
      p["Wrel1"], p["brel1"].reshape(1, H), p["Wroot1"], p["W1"], p["b1"].reshape(1, H),
      p["Wrel2"], p["brel2"].reshape(1, H), p["Wroot2"], p["W2"], p["b2"].reshape(1, H),
      p["Wcat"], p["bcat"].reshape(1, H), p["Wl0"], p["bl0"].reshape(1, H),
      p["Wl1"], p["bl1"].reshape(1, H))


def _b3_body(h_ref, bidc_ref, gsum_ref, gsum2_ref, gcnt_ref,
             nw_ref, nb_ref, ms_ref, wfin_ref, bfin_ref, o_ref):
    cnt = jnp.maximum(gcnt_ref[...], 1.0)
    mean = gsum_ref[...] / cnt
    meansq = gsum2_ref[...] / cnt
    ms = ms_ref[...]
    var = meansq - mean * mean * ms * (2.0 - ms)
    std = jnp.sqrt(var + 1e-5)
    idc = bidc_ref[...]
    ohc = (lax.broadcasted_iota(jnp.int32, (NB, NG), 1) == idc).astype(jnp.float32)
    cen = h_ref[...] - _dot(ohc, mean) * ms
    hn = nw_ref[...] * cen / _dot(ohc, std) + nb_ref[...]
    o_ref[...] = _dot(hn, wfin_ref[...]) + bfin_ref[...]


def _b3(hpre, bid_col, gsum, gsum2, gcnt, p):
    return pl.pallas_call(
        _b3_body,
        grid=(N // NB,),
        in_specs=[
            pl.BlockSpec((NB, H), lambda i: (i, 0)),
            pl.BlockSpec((NB, 1), lambda i: (i, 0)),
            pl.BlockSpec((NG, H), lambda i: (0, 0)),
            pl.BlockSpec((NG, H), lambda i: (0, 0)),
            pl.BlockSpec((NG, H), lambda i: (0, 0)),
            pl.BlockSpec((1, H), lambda i: (0, 0)),
            pl.BlockSpec((1, H), lambda i: (0, 0)),
            pl.BlockSpec((1, H), lambda i: (0, 0)),
            pl.BlockSpec((H, H), lambda i: (0, 0)),
            pl.BlockSpec((1, H), lambda i: (0, 0)),
        ],
        out_specs=pl.BlockSpec((NB, H), lambda i: (i, 0)),
        out_shape=jax.ShapeDtypeStruct((N, H), jnp.float32),
    )(hpre, bid_col, gsum, gsum2, gcnt,
      p["norm_w"].reshape(1, H), p["norm_b"].reshape(1, H),
      p["norm_ms"].reshape(1, H), p["Wfin"], p["bfin"].reshape(1, H))


def kernel(x, feature1, feature2, edge_index, batch, params):
    p = params
    ei = edge_index.astype(jnp.int32)
    src = ei[0].reshape(2, 16, NCHUNK, CHUNK)
    dst = ei[1].reshape(2, 16, NCHUNK, CHUNK)
    bid = batch.astype(jnp.int32)
    bid_row = bid.reshape(N // NB, 1, NB)
    bid_col = bid.reshape(N, 1)

    xp = _xprime(x, p["W_lin"], p["b_lin"].reshape(1, H))
    zeros = jnp.zeros((NPAD, H), jnp.float32)
    f1T, f2T = feature1.T, feature2.T
    fA = _edgefeat(0, f1T, f2T, p["Wf1a"], p["Wf1b"], p["Wf2a"], p["Wf2b"])
    aggA = _sc_agg(0, xp, fA, src, dst, zeros)
    fB = _edgefeat(1, f1T, f2T, p["Wf1a"], p["Wf1b"], p["Wf2a"], p["Wf2b"])
    aggB = _sc_agg(1, xp, fB, src, dst, zeros)
    hpre, gsum, gsum2, gcnt = _b1(aggA[:, :N], aggB[:, :N], xp, bid_row, p)
    return _b3(hpre, bid_col, gsum, gsum2, gcnt, p)


# speedup vs baseline: 2.2770x; 1.0102x over previous
import functools

import jax
import jax.numpy as jnp
from jax import lax
from jax.experimental import pallas as pl
from jax.experimental.pallas import tpu as pltpu
from jax.experimental.pallas import tpu_sc as plsc

N = 10000
E = 320000
H = 128
NG = 64
F1 = 54
F2 = 18
MID = 64

NB = 2000
EB = 6400
NPAD = 10240
ROWS_PER_TILE = NPAD // 16
EHALF = E // 2
EDGES_PER_TILE = EHALF // 16
CHUNK = 40
NCHUNK = EDGES_PER_TILE // CHUNK

_P = jax.lax.Precision.HIGHEST


def _swish(v):
    return v * jax.nn.sigmoid(v)


def _dot(a, b):
    return jnp.dot(a, b, precision=_P, preferred_element_type=jnp.float32)


def _xprime_body(x_ref, w_ref, b_ref, o_ref):
    o_ref[...] = _swish(_dot(x_ref[...], w_ref[...]) + b_ref[...])


def _xprime(x, W, b):
    return pl.pallas_call(
        _xprime_body,
        grid=(N // NB,),
        in_specs=[
            pl.BlockSpec((NB, H), lambda i: (i, 0)),
            pl.BlockSpec((H, H), lambda i: (0, 0)),
            pl.BlockSpec((1, H), lambda i: (0, 0)),
        ],
        out_specs=pl.BlockSpec((NB, H), lambda i: (i, 0)),
        out_shape=jax.ShapeDtypeStruct((N, H), jnp.float32),
    )(x, W, b)


def _dotT(aT, b):
    return lax.dot_general(aT, b, (((0,), (0,)), ((), ())),
                           precision=_P, preferred_element_type=jnp.float32)


def _edgefeat_body(f1T_ref, f2T_ref, wa1_ref, wb1_ref, wa2_ref, wb2_ref, o_ref):
    w1 = _dot(wa1_ref[...], wb1_ref[...])
    w2 = _dot(wa2_ref[...], wb2_ref[...])
    o_ref[0] = _dotT(f1T_ref[...], w1)
    o_ref[1] = _dotT(f2T_ref[...], w2)


def _edgefeat(half, feature1T, feature2T, Wf1a, Wf1b, Wf2a, Wf2b):
    off = half * (EHALF // EB)
    return pl.pallas_call(
        _edgefeat_body,
        grid=(EHALF // EB,),
        in_specs=[
            pl.BlockSpec((F1, EB), lambda i: (0, i + off)),
            pl.BlockSpec((F2, EB), lambda i: (0, i + off)),
            pl.BlockSpec((F1, MID), lambda i: (0, 0)),
            pl.BlockSpec((MID, H), lambda i: (0, 0)),
            pl.BlockSpec((F2, MID), lambda i: (0, 0)),
            pl.BlockSpec((MID, H), lambda i: (0, 0)),
        ],
        out_specs=pl.BlockSpec((2, EB, H), lambda i: (0, i, 0)),
        out_shape=jax.ShapeDtypeStruct((2, EHALF, H), jnp.float32),
    )(feature1T, feature2T, Wf1a, Wf1b, Wf2a, Wf2b)


def _sc_agg(half, xp, f, src4, dst4, zeros):
    mesh = plsc.VectorSubcoreMesh(core_axis_name="c", subcore_axis_name="s")

    @functools.partial(
        pl.kernel,
        mesh=mesh,
        out_type=jax.ShapeDtypeStruct((2, NPAD, H), jnp.float32),
        scratch_types=[
            pltpu.VMEM((CHUNK,), jnp.int32),
            pltpu.VMEM((CHUNK,), jnp.int32),
            pltpu.VMEM((CHUNK,), jnp.int32),
            pltpu.VMEM((CHUNK,), jnp.int32),
            pltpu.VMEM((CHUNK, H), jnp.float32),
            pltpu.VMEM((CHUNK, H), jnp.float32),
            pltpu.VMEM((CHUNK, H), jnp.float32),
            pltpu.VMEM((CHUNK, H), jnp.float32),
            pltpu.VMEM((CHUNK, H), jnp.float32),
            pltpu.VMEM((CHUNK, H), jnp.float32),
            pltpu.VMEM_SHARED((NPAD, H), jnp.float32),
            pltpu.SemaphoreType.DMA,
            pltpu.SemaphoreType.DMA,
            pltpu.SemaphoreType.DMA,
            pltpu.SemaphoreType.DMA,
            pltpu.SemaphoreType.DMA,
            pltpu.SemaphoreType.DMA,
            pltpu.SemaphoreType.DMA,
            pltpu.SemaphoreType.DMA,
            pltpu.SemaphoreType.DMA,
            pltpu.SemaphoreType.DMA,
        ],
    )
    def k(x_hbm, f_hbm, src_hbm, dst_hbm, z_hbm, agg_hbm,
          sv0, sv1, dv0, dv1, xr0, xr1, fb0, fb1, pr0, pr1, agg_sh,
          sG0, sG1, sF0, sF1, sSI0, sSI1, sDI0, sDI1, sS0, sS1):
        c = lax.axis_index("c")
        s = lax.axis_index("s")
        sv = (sv0, sv1)
        dv = (dv0, dv1)
        xr = (xr0, xr1)
        fb = (fb0, fb1)
        pr = (pr0, pr1)
        sG = (sG0, sG1)
        sF = (sF0, sF1)
        sSI = (sSI0, sSI1)
        sDI = (sDI0, sDI1)
        sS = (sS0, sS1)
        nbase = s * ROWS_PER_TILE
        ebase = s * EDGES_PER_TILE

        def start_data(j, b):
            pltpu.async_copy(x_hbm.at[sv[b]], xr[b], sG[b])
            pltpu.async_copy(f_hbm.at[c, pl.ds(ebase + j * CHUNK, CHUNK)],
                             fb[b], sF[b])

        for b in range(2):
            pltpu.sync_copy(src_hbm.at[half, s, b], sv[b])
            pltpu.sync_copy(dst_hbm.at[half, s, b], dv[b])
            start_data(b, b)

        pltpu.sync_copy(z_hbm.at[pl.ds(nbase, ROWS_PER_TILE)],
                        agg_sh.at[pl.ds(nbase, ROWS_PER_TILE)])
        plsc.subcore_barrier()

        def pair(g, carry):
            for b in range(2):
                j = g * 2 + b
                pltpu.make_async_copy(x_hbm.at[sv[b]], xr[b], sG[b]).wait()

                @pl.when(g < NCHUNK // 2 - 1)
                def _():
                    pltpu.async_copy(src_hbm.at[half, s, j + 2], sv[b], sSI[b])

                pltpu.make_async_copy(
                    f_hbm.at[c, pl.ds(ebase + j * CHUNK, CHUNK)], fb[b],
                    sF[b]).wait()

                @plsc.parallel_loop(0, CHUNK, unroll=4)
                def _(r):
                    for kk in range(H // 16):
                        sl = pl.ds(kk * 16, 16)
                        pr[b][r, sl] = fb[b][r, sl] * xr[b][r, sl]

                if b == 0:
                    @pl.when(g >= 1)
                    def _():
                        pltpu.make_async_copy(pr[1 - b], agg_sh.at[dv[1 - b]],
                                              sS[1 - b]).wait()
                else:
                    pltpu.make_async_copy(pr[1 - b], agg_sh.at[dv[1 - b]],
                                          sS[1 - b]).wait()

                @pl.when(g >= 1)
                def _():
                    pltpu.make_async_copy(dst_hbm.at[half, s, j], dv[b],
                                          sDI[b]).wait()

                pltpu.async_copy(pr[b], agg_sh.at[dv[b]], sS[b], add=True)

                @pl.when((g >= 1) if b == 0 else (g < NCHUNK // 2 - 1))
                def _():
                    pltpu.async_copy(dst_hbm.at[half, s, j + 1], dv[1 - b],
                                     sDI[1 - b])

                @pl.when(g < NCHUNK // 2 - 1)
                def _():
                    pltpu.make_async_copy(src_hbm.at[half, s, j + 2], sv[b],
                                          sSI[b]).wait()
                    start_data(j + 2, b)
            return carry

        lax.fori_loop(0, NCHUNK // 2, pair, 0)
        pltpu.make_async_copy(pr[1], agg_sh.at[dv[1]], sS[1]).wait()
        plsc.subcore_barrier()
        pltpu.sync_copy(agg_sh.at[pl.ds(nbase, ROWS_PER_TILE)],
                        agg_hbm.at[c, pl.ds(nbase, ROWS_PER_TILE)])

    return k(xp, f, src4, dst4, zeros)


def _b1_body(aggA_ref, aggB_ref, x_ref, bid_ref,
             wrel1_ref, brel1_ref, wroot1_ref, w1_ref, b1_ref,
             wrel2_ref, brel2_ref, wroot2_ref, w2_ref, b2_ref,
             wcat_ref, bcat_ref, wl0_ref, bl0_ref, wl1_ref, bl1_ref,
             hpre_ref, gsum_ref, gsum2_ref, gcnt_ref):
    xb = x_ref[...]
    agg1 = aggA_ref[0] + aggB_ref[0]
    agg2 = aggA_ref[1] + aggB_ref[1]
    h1 = _dot(agg1, wrel1_ref[...]) + brel1_ref[...] + _dot(xb, wroot1_ref[...])
    h1 = _swish(_dot(h1, w1_ref[...]) + b1_ref[...])
    h2 = _dot(agg2, wrel2_ref[...]) + brel2_ref[...] + _dot(xb, wroot2_ref[...])
    h2 = _swish(_dot(h2, w2_ref[...]) + b2_ref[...])
    h = _dot(h1, wcat_ref[...][:H]) + _dot(h2, wcat_ref[...][H:]) + bcat_ref[...] + xb
    h = _swish(_dot(h, wl0_ref[...]) + bl0_ref[...]) + h
    h = _swish(_dot(h, wl1_ref[...]) + bl1_ref[...]) + h
    hpre_ref[...] = h
    ids = bid_ref[0]
    oh = (lax.broadcasted_iota(jnp.int32, (NG, NB), 0) == ids).astype(jnp.float32)
    psum = _dot(oh, h)
    psum2 = _dot(oh, h * h)
    pcnt = jnp.broadcast_to(jnp.sum(oh, axis=1, keepdims=True), (NG, H))

    @pl.when(pl.program_id(0) == 0)
    def _():
        gsum_ref[...] = psum
        gsum2_ref[...] = psum2
        gcnt_ref[...] = pcnt

    @pl.when(pl.program_id(0) != 0)
    def _():
        gsum_ref[...] += psum
        gsum2_ref[...] += psum2
        gcnt_ref[...] += pcnt


def _b1(aggA, aggB, xp, bid_row, p):
    wspec = pl.BlockSpec((H, H), lambda i: (0, 0))
    bspec = pl.BlockSpec((1, H), lambda i: (0, 0))
    return pl.pallas_call(
        _b1_body,
        grid=(N // NB,),
        in_specs=[
            pl.BlockSpec((2, NB, H), lambda i: (0, i, 0)),
            pl.BlockSpec((2, NB, H), lambda i: (0, i, 0)),
            pl.BlockSpec((NB, H), lambda i: (i, 0)),
            pl.BlockSpec((1, 1, NB), lambda i: (i, 0, 0)),
            wspec, bspec, wspec, wspec, bspec,
            wspec, bspec, wspec, wspec, bspec,
            pl.BlockSpec((2 * H, H), lambda i: (0, 0)), bspec,
            wspec, bspec, wspec, bspec,
        ],
        out_specs=[
            pl.BlockSpec((NB, H), lambda i: (i, 0)),
            pl.BlockSpec((NG, H), lambda i: (0, 0)),
            pl.BlockSpec((NG, H), lambda i: (0, 0)),
            pl.BlockSpec((NG, H), lambda i: (0, 0)),
        ],
        out_shape=[
            jax.ShapeDtypeStruct((N, H), jnp.float32),
            jax.ShapeDtypeStruct((NG, H), jnp.float32),
            jax.ShapeDtypeStruct((NG, H), jnp.float32),
            jax.ShapeDtypeStruct((NG, H), jnp.float32),
        ],
    )(aggA, aggB, xp, bid_row,
      p["Wrel1"], p["brel1"].reshape(1, H), p["Wroot1"], p["W1"], p["b1"].reshape(1, H),
      p["Wrel2"], p["brel2"].reshape(1, H), p["Wroot2"], p["W2"], p["b2"].reshape(1, H),
      p["Wcat"], p["bcat"].reshape(1, H), p["Wl0"], p["bl0"].reshape(1, H),
      p["Wl1"], p["bl1"].reshape(1, H))


def _b3_body(h_ref, bidc_ref, gsum_ref, gsum2_ref, gcnt_ref,
             nw_ref, nb_ref, ms_ref, wfin_ref, bfin_ref, o_ref):
    cnt = jnp.maximum(gcnt_ref[...], 1.0)
    mean = gsum_ref[...] / cnt
    meansq = gsum2_ref[...] / cnt
    ms = ms_ref[...]
    var = meansq - mean * mean * ms * (2.0 - ms)
    std = jnp.sqrt(var + 1e-5)
    idc = bidc_ref[...]
    ohc = (lax.broadcasted_iota(jnp.int32, (NB, NG), 1) == idc).astype(jnp.float32)
    cen = h_ref[...] - _dot(ohc, mean) * ms
    hn = nw_ref[...] * cen / _dot(ohc, std) + nb_ref[...]
    o_ref[...] = _dot(hn, wfin_ref[...]) + bfin_ref[...]


def _b3(hpre, bid_col, gsum, gsum2, gcnt, p):
    return pl.pallas_call(
        _b3_body,
        grid=(N // NB,),
        in_specs=[
            pl.BlockSpec((NB, H), lambda i: (i, 0)),
            pl.BlockSpec((NB, 1), lambda i: (i, 0)),
            pl.BlockSpec((NG, H), lambda i: (0, 0)),
            pl.BlockSpec((NG, H), lambda i: (0, 0)),
            pl.BlockSpec((NG, H), lambda i: (0, 0)),
            pl.BlockSpec((1, H), lambda i: (0, 0)),
            pl.BlockSpec((1, H), lambda i: (0, 0)),
            pl.BlockSpec((1, H), lambda i: (0, 0)),
            pl.BlockSpec((H, H), lambda i: (0, 0)),
            pl.BlockSpec((1, H), lambda i: (0, 0)),
        ],
        out_specs=pl.BlockSpec((NB, H), lambda i: (i, 0)),
        out_shape=jax.ShapeDtypeStruct((N, H), jnp.float32),
    )(hpre, bid_col, gsum, gsum2, gcnt,
      p["norm_w"].reshape(1, H), p["norm_b"].reshape(1, H),
      p["norm_ms"].reshape(1, H), p["Wfin"], p["bfin"].reshape(1, H))


def kernel(x, feature1, feature2, edge_index, batch, params):
    p = params
    ei = edge_index.astype(jnp.int32)
    src = ei[0].reshape(2, 16, NCHUNK, CHUNK)
    dst = ei[1].reshape(2, 16, NCHUNK, CHUNK)
    bid = batch.astype(jnp.int32)
    bid_row = bid.reshape(N // NB, 1, NB)
    bid_col = bid.reshape(N, 1)

    xp = _xprime(x, p["W_lin"], p["b_lin"].reshape(1, H))
    zeros = jnp.zeros((NPAD, H), jnp.float32)
    f1T, f2T = feature1.T, feature2.T
    fA = _edgefeat(0, f1T, f2T, p["Wf1a"], p["Wf1b"], p["Wf2a"], p["Wf2b"])
    aggA = _sc_agg(0, xp, fA, src, dst, zeros)
    fB = _edgefeat(1, f1T, f2T, p["Wf1a"], p["Wf1b"], p["Wf2a"], p["Wf2b"])
    aggB = _sc_agg(1, xp, fB, src, dst, zeros)
    hpre, gsum, gsum2, gcnt = _b1(aggA, aggB, xp, bid_row, p)
    return _b3(hpre, bid_col, gsum, gsum2, gcnt, p)
